# baseline (speedup 1.0000x reference)
import functools

import jax
import jax.numpy as jnp
from jax import lax
from jax.experimental import pallas as pl
from jax.experimental.pallas import tpu as pltpu
from jax.experimental.pallas import tpu_sc as plsc

N = 10000
E = 160000
D = 256
H = 128
NLAYER = 5
NC = 2
NS = 16
GB = 128
NGB = 40
NPASS = 3
NGA = 40
PASS_ROWS = 3456
STRIPE2 = 216
ACC2_R = NS * STRIPE2 + 16
PAD2 = ACC2_R - 8
NCOMBO = 18


def _zero_rows(zb, nrows, ncol16):
    def body(i, carry):
        for j in range(ncol16):
            zb[i, pl.ds(j * 16, 16)] = jnp.zeros((16,), jnp.float32)
        return carry
    lax.fori_loop(0, nrows, body, 0)


def _sc_scatter_body(h_hbm, ee_hbm, src_hbm, dst_hbm, c_hbm, out_hbm,
                     src_v, dst_v, c_v, rows_v, ee_v, zb, acc, sem):
    cid = lax.axis_index("c")
    sid = lax.axis_index("s")
    _zero_rows(zb, STRIPE2, H // 16)
    base = sid * STRIPE2
    for p in range(NPASS):
        pltpu.sync_copy(src_hbm.at[p].at[sid], src_v)
        pltpu.sync_copy(dst_hbm.at[p].at[sid], dst_v)
        pltpu.sync_copy(c_hbm.at[p].at[sid], c_v)
        pltpu.sync_copy(zb, acc.at[pl.ds(base, STRIPE2)])
        plsc.subcore_barrier()

        def step(j, carry):
            cp = pltpu.async_copy(h_hbm.at[cid].at[src_v.at[j]], rows_v, sem)
            cp.wait()
            cp2 = pltpu.async_copy(ee_hbm.at[cid].at[c_v.at[j]], ee_v, sem)
            cp2.wait()

            def addrow(i, c2):
                for q in range(H // 16):
                    sl = pl.ds(q * 16, 16)
                    rows_v[i, sl] = rows_v[i, sl] + ee_v[i, sl]
                return c2
            lax.fori_loop(0, GB, addrow, 0)
            pltpu.sync_copy(rows_v, acc.at[dst_v.at[j]], add=True)
            return carry
        lax.fori_loop(0, NGB, step, 0)
        plsc.subcore_barrier()

        rows_p = min(PASS_ROWS, N - p * PASS_ROWS)
        full_tiles = rows_p // STRIPE2
        rem = rows_p - full_tiles * STRIPE2

        @pl.when(sid < full_tiles)
        def _():
            pltpu.sync_copy(
                acc.at[pl.ds(base, STRIPE2)],
                out_hbm.at[cid].at[pl.ds(p * PASS_ROWS + base, STRIPE2)])

        if rem:
            @pl.when(sid == full_tiles)
            def _():
                pltpu.sync_copy(
                    acc.at[pl.ds(base, rem)],
                    out_hbm.at[cid].at[pl.ds(p * PASS_ROWS + base, rem)])


@functools.cache
def _get_sc_kernels():
    mesh = plsc.VectorSubcoreMesh(
        core_axis_name="c", subcore_axis_name="s",
        num_cores=NC, num_subcores=NS)
    scatter = pl.kernel(
        _sc_scatter_body,
        out_type=jax.ShapeDtypeStruct((NC, N, H), jnp.float32),
        mesh=mesh,
        scratch_types=[
            pltpu.VMEM((NGB, GB), jnp.int32),
            pltpu.VMEM((NGB, GB), jnp.int32),
            pltpu.VMEM((NGB, GB), jnp.int32),
            pltpu.VMEM((GB, H), jnp.float32),
            pltpu.VMEM((GB, H), jnp.float32),
            pltpu.VMEM((STRIPE2, H), jnp.float32),
            pltpu.VMEM_SHARED((ACC2_R, H), jnp.float32),
            pltpu.SemaphoreType.DMA,
        ],
    )
    return scatter


TR = 1000
NROW_T = N // TR


def _mlp_body(scat, w1, b1, w2, b2, x_out, stats_out, stats_scr):
    i = pl.program_id(0)
    aggr = jnp.concatenate([scat[0], scat[1]], axis=1)
    hmid = jnp.maximum(
        jnp.dot(aggr, w1[...], preferred_element_type=jnp.float32,
                precision=lax.Precision.HIGHEST) + b1[...],
        0.0)
    xt = jnp.dot(hmid, w2[...], preferred_element_type=jnp.float32,
                 precision=lax.Precision.HIGHEST) + b2[...]
    x_out[...] = xt

    @pl.when(i == 0)
    def _():
        stats_scr[...] = jnp.zeros((8, D), jnp.float32)

    stats_scr[pl.ds(0, 1)] = stats_scr[pl.ds(0, 1)] + jnp.sum(
        xt, axis=0, keepdims=True)
    stats_scr[pl.ds(1, 1)] = stats_scr[pl.ds(1, 1)] + jnp.sum(
        xt * xt, axis=0, keepdims=True)
    stats_out[...] = stats_scr[...]


_mlp_call = pl.pallas_call(
    _mlp_body,
    grid=(NROW_T,),
    in_specs=[
        pl.BlockSpec((NC, TR, H), lambda i: (0, i, 0)),
        pl.BlockSpec((D, 2 * D), lambda i: (0, 0)),
        pl.BlockSpec((1, 2 * D), lambda i: (0, 0)),
        pl.BlockSpec((2 * D, D), lambda i: (0, 0)),
        pl.BlockSpec((1, D), lambda i: (0, 0)),
    ],
    out_specs=[
        pl.BlockSpec((TR, D), lambda i: (i, 0)),
        pl.BlockSpec((8, D), lambda i: (0, 0)),
    ],
    out_shape=[
        jax.ShapeDtypeStruct((N, D), jnp.float32),
        jax.ShapeDtypeStruct((8, D), jnp.float32),
    ],
    scratch_shapes=[pltpu.VMEM((8, D), jnp.float32)],
)


def _make_norm_call(last):
    def body(x, stats, gamma, beta, out):
        mean = stats[pl.ds(0, 1)] * (1.0 / N)
        ex2 = stats[pl.ds(1, 1)] * (1.0 / N)
        var = ex2 - mean * mean
        inv = lax.rsqrt(var + 1e-5)
        hh = (x[...] - mean) * inv * gamma[...] + beta[...]
        if not last:
            hh = jnp.maximum(hh, 0.0)
        if last:
            out[...] = hh
        else:
            out[0] = hh[:, :H]
            out[1] = hh[:, H:]

    out_spec = (pl.BlockSpec((TR, D), lambda i: (i, 0)) if last
                else pl.BlockSpec((NC, TR, H), lambda i: (0, i, 0)))
    out_shape = (jax.ShapeDtypeStruct((N, D), jnp.float32) if last
                 else jax.ShapeDtypeStruct((NC, N, H), jnp.float32))
    return pl.pallas_call(
        body,
        grid=(NROW_T,),
        in_specs=[
            pl.BlockSpec((TR, D), lambda i: (i, 0)),
            pl.BlockSpec((8, D), lambda i: (0, 0)),
            pl.BlockSpec((1, D), lambda i: (0, 0)),
            pl.BlockSpec((1, D), lambda i: (0, 0)),
        ],
        out_specs=out_spec,
        out_shape=out_shape,
    )


_norm_mid = _make_norm_call(False)
_norm_last = _make_norm_call(True)


def _embed_body(xb, e1, e2, out):
    a0 = xb[:, 0:1]
    a1 = xb[:, 1:2]
    oh1 = (a0 == lax.broadcasted_iota(jnp.int32, (TR, 120), 1)
           ).astype(jnp.float32)
    oh2 = (a1 == lax.broadcasted_iota(jnp.int32, (TR, 8), 1)
           ).astype(jnp.float32)
    hh = (jnp.dot(oh1, e1[...], preferred_element_type=jnp.float32,
                      precision=lax.Precision.HIGHEST)
          + jnp.dot(oh2, e2[...], preferred_element_type=jnp.float32,
                      precision=lax.Precision.HIGHEST))
    out[0] = hh[:, :H]
    out[1] = hh[:, H:]


_embed_call = pl.pallas_call(
    _embed_body,
    grid=(NROW_T,),
    in_specs=[
        pl.BlockSpec((TR, 2), lambda i: (i, 0)),
        pl.BlockSpec((120, D), lambda i: (0, 0)),
        pl.BlockSpec((8, D), lambda i: (0, 0)),
    ],
    out_specs=pl.BlockSpec((NC, TR, H), lambda i: (0, i, 0)),
    out_shape=jax.ShapeDtypeStruct((NC, N, H), jnp.float32),
)


def kernel(x, edge_index, edge_attr, x_emb1, x_emb2, edge_emb1, edge_emb2,
           W1, b1, W2, b2, gamma, beta):
    f32 = jnp.float32
    i32 = jnp.int32
    src = edge_index[0]
    dst = edge_index[1]

    loop = jnp.arange(N, dtype=i32)
    srcf = jnp.concatenate([src, loop])
    dstf = jnp.concatenate([dst, loop])
    combo = edge_attr[:, 0] * 3 + edge_attr[:, 1]
    cf = jnp.concatenate([combo, jnp.full((N,), NCOMBO, i32)])
    ef = E + N

    p_of = dstf // PASS_ROWS
    local = dstf - p_of * PASS_ROWS
    t_of = local // STRIPE2
    bucket = p_of * NS + t_of
    order = jnp.argsort(bucket, stable=True)
    bsrc = srcf[order]
    bloc = local[order]
    bc = cf[order]
    cnt = jnp.bincount(bucket, length=NPASS * NS)
    start = jnp.concatenate([jnp.zeros((1,), cnt.dtype), jnp.cumsum(cnt)[:-1]])
    cap = NGB * GB
    pos = jnp.arange(cap, dtype=i32)
    idx = start[:, None].astype(i32) + pos[None, :]
    valid = pos[None, :] < cnt[:, None]
    ii = jnp.clip(idx, 0, ef - 1)
    srcp = jnp.where(valid, bsrc[ii], 0).reshape(NPASS, NS, NGB, GB)
    dstp = jnp.where(valid, bloc[ii], PAD2).reshape(NPASS, NS, NGB, GB)
    cp = jnp.where(valid, bc[ii], 31).reshape(NPASS, NS, NGB, GB)

    cty = jnp.arange(NCOMBO, dtype=i32) // 3
    cdr = jnp.arange(NCOMBO, dtype=i32) % 3
    tbl = edge_emb1[:, cty] + edge_emb2[:, cdr]
    tself = edge_emb1[:, 4] + edge_emb2[:, 0]
    tbl = jnp.concatenate([tbl, tself[:, None, :]], axis=1)
    tbl = jnp.pad(tbl, ((0, 0), (0, 32 - 19), (0, 0)))
    eetabs = jnp.stack([tbl[:, :, :H], tbl[:, :, H:]], axis=1)

    e2pad = jnp.pad(x_emb2, ((0, 8 - x_emb2.shape[0]), (0, 0)))

    sc_scatter = _get_sc_kernels()
    h = _embed_call(x, x_emb1, e2pad)

    for l in range(NLAYER):
        scat = sc_scatter(h, eetabs[l], srcp, dstp, cp)
        xx, stats = _mlp_call(scat, W1[l], b1[l][None], W2[l], b2[l][None])
        if l == NLAYER - 1:
            h = _norm_last(xx, stats, gamma[l][None], beta[l][None])
        else:
            h = _norm_mid(xx, stats, gamma[l][None], beta[l][None])
    return h
